# baseline (speedup 1.0000x reference)
import functools

import jax
import jax.numpy as jnp
from jax import lax
from jax.experimental import pallas as pl
from jax.experimental.pallas import tpu as pltpu
from jax.experimental.pallas import tpu_sc as plsc

N = 10000
D = 128
E = 320000
EPS = 1e-5

NC = 2
NS = 16
NW = NC * NS
CH = 128
K_AGG = 80
K_DEG = 160
E_PAD = NW * K_AGG * CH
ROWS_SH = 10112
RPT = ROWS_SH // NS


def _sc_degrees(deg_idx, ones_hbm, zeros_hbm):
  mesh = plsc.VectorSubcoreMesh(core_axis_name="c", subcore_axis_name="s")

  @functools.partial(
      pl.kernel,
      out_type=jax.ShapeDtypeStruct((NC, ROWS_SH, D), jnp.float32),
      mesh=mesh,
      scratch_types=[
          pltpu.VMEM((K_DEG, CH), jnp.int32),
          pltpu.VMEM((CH, D), jnp.float32),
          pltpu.VMEM_SHARED((ROWS_SH, D), jnp.float32),
      ],
  )
  def k(idx_hbm, ones_h, zeros_h, out_hbm, idx_v, ones_v, deg_sh):
    c = lax.axis_index("c")
    s = lax.axis_index("s")
    pltpu.sync_copy(idx_hbm.at[c, s], idx_v)
    pltpu.sync_copy(ones_h, ones_v)
    pltpu.sync_copy(zeros_h, deg_sh.at[pl.ds(s * RPT, RPT)])
    plsc.subcore_barrier()

    @pl.loop(0, K_DEG)
    def _(j):
      pltpu.sync_copy(ones_v, deg_sh.at[idx_v.at[j]], add=True)

    plsc.subcore_barrier()
    pltpu.sync_copy(deg_sh.at[pl.ds(s * RPT, RPT)],
                    out_hbm.at[c, pl.ds(s * RPT, RPT)])

  return k(deg_idx, ones_hbm, zeros_hbm)


def _sc_aggregate(h_pad, src_t, dst_t, zeros_hbm):
  mesh = plsc.VectorSubcoreMesh(core_axis_name="c", subcore_axis_name="s")

  @functools.partial(
      pl.kernel,
      out_type=jax.ShapeDtypeStruct((NC, ROWS_SH, D), jnp.float32),
      mesh=mesh,
      scratch_types=[
          pltpu.VMEM((K_AGG // 2, CH), jnp.int32),
          pltpu.VMEM((K_AGG // 2, CH), jnp.int32),
          pltpu.VMEM((CH, D), jnp.float32),
          pltpu.VMEM((CH, D), jnp.float32),
          pltpu.VMEM_SHARED((ROWS_SH, D), jnp.float32),
          pltpu.SemaphoreType.DMA,
          pltpu.SemaphoreType.DMA,
      ],
  )
  def k(h_hbm, src_h, dst_h, z_h, out_hbm, sidx_v, didx_v, rows0, rows1,
        agg_sh, g0, g1):
    c = lax.axis_index("c")
    s = lax.axis_index("s")
    wid = c * NS + s
    half = K_AGG // 2
    pltpu.sync_copy(z_h, agg_sh.at[pl.ds(s * RPT, RPT)])
    plsc.subcore_barrier()

    def wait_gather(buf, sem):
      pltpu.make_async_copy(h_hbm.at[sidx_v.at[0]], buf, sem).wait()

    for hf in range(2):
      pltpu.sync_copy(src_h.at[wid, pl.ds(hf * half, half)], sidx_v)
      pltpu.sync_copy(dst_h.at[wid, pl.ds(hf * half, half)], didx_v)
      pltpu.async_copy(h_hbm.at[sidx_v.at[0]], rows0, g0)

      @pl.loop(0, half - 2, step=2)
      def _(j):
        wait_gather(rows0, g0)
        pltpu.async_copy(h_hbm.at[sidx_v.at[j + 1]], rows1, g1)
        pltpu.sync_copy(rows0, agg_sh.at[didx_v.at[j]], add=True)
        wait_gather(rows1, g1)
        pltpu.async_copy(h_hbm.at[sidx_v.at[j + 2]], rows0, g0)
        pltpu.sync_copy(rows1, agg_sh.at[didx_v.at[j + 1]], add=True)

      wait_gather(rows0, g0)
      pltpu.async_copy(h_hbm.at[sidx_v.at[half - 1]], rows1, g1)
      pltpu.sync_copy(rows0, agg_sh.at[didx_v.at[half - 2]], add=True)
      wait_gather(rows1, g1)
      pltpu.sync_copy(rows1, agg_sh.at[didx_v.at[half - 1]], add=True)

    plsc.subcore_barrier()
    pltpu.sync_copy(agg_sh.at[pl.ds(s * RPT, RPT)],
                    out_hbm.at[c, pl.ds(s * RPT, RPT)])

  return k(h_pad, src_t, dst_t, zeros_hbm)


def _tc_matmul(x_pad, w1):
  def body(x_ref, w_ref, o_ref):
    o_ref[...] = jnp.dot(x_ref[...], w_ref[...],
                         preferred_element_type=jnp.float32)

  return pl.pallas_call(
      body,
      out_shape=jax.ShapeDtypeStruct((ROWS_SH, D), jnp.float32),
  )(x_pad, w1)


def _tc_scale_l1(u, deg):
  def body(u_ref, d_ref, h_ref, s_ref):
    out_s = lax.rsqrt(jnp.maximum(d_ref[0, :, 0:1], 1.0))
    in_s = lax.rsqrt(jnp.maximum(d_ref[1, :, 0:1], 1.0))
    h_ref[...] = u_ref[...] * out_s
    s_ref[...] = jnp.concatenate(
        [jnp.broadcast_to(out_s, (ROWS_SH, 8)),
         jnp.broadcast_to(in_s, (ROWS_SH, 8))], axis=1)

  return pl.pallas_call(
      body,
      out_shape=(jax.ShapeDtypeStruct((ROWS_SH, D), jnp.float32),
                 jax.ShapeDtypeStruct((ROWS_SH, 16), jnp.float32)),
  )(u, deg)


def _tc_mid(p, scales, b1, g1, be1, w2):
  def body(p_ref, s_ref, b_ref, g_ref, be_ref, w_ref, o_ref):
    agg = p_ref[0, :N, :] + p_ref[1, :N, :]
    agg = agg * s_ref[0:N, 8:9] + b_ref[...]
    a = jnp.where(agg > 0, agg, jnp.exp(agg) - 1.0)
    mean = jnp.mean(a, axis=0, keepdims=True)
    var = jnp.mean((a - mean) ** 2, axis=0, keepdims=True)
    h1 = (a - mean) * lax.rsqrt(var + EPS) * g_ref[...] + be_ref[...]
    h2 = jnp.dot(h1, w_ref[...], preferred_element_type=jnp.float32)
    o_ref[0:N, :] = h2 * s_ref[0:N, 0:1]
    o_ref[N:ROWS_SH, :] = jnp.zeros((ROWS_SH - N, D), jnp.float32)

  return pl.pallas_call(
      body,
      out_shape=jax.ShapeDtypeStruct((ROWS_SH, D), jnp.float32),
  )(p, scales, b1, g1, be1, w2)


def _tc_post(p, scales, b2, g2, be2):
  def body(p_ref, s_ref, b_ref, g_ref, be_ref, o_ref):
    agg = p_ref[0, :N, :] + p_ref[1, :N, :]
    agg = agg * s_ref[0:N, 8:9] + b_ref[...]
    a = jnp.where(agg > 0, agg, jnp.exp(agg) - 1.0)
    mean = jnp.mean(a, axis=0, keepdims=True)
    var = jnp.mean((a - mean) ** 2, axis=0, keepdims=True)
    o_ref[...] = (a - mean) * lax.rsqrt(var + EPS) * g_ref[...] + be_ref[...]

  return pl.pallas_call(
      body,
      out_shape=jax.ShapeDtypeStruct((N, D), jnp.float32),
  )(p, scales, b2, g2, be2)


@jax.jit
def kernel(features, edge_index, W1, b1, gamma1, beta1, W2, b2, gamma2,
           beta2):
  src = edge_index[0].astype(jnp.int32)
  dst = edge_index[1].astype(jnp.int32)
  pad = jnp.full((E_PAD - E,), N, jnp.int32)
  src_p = jnp.concatenate([src, pad])
  dst_p = jnp.concatenate([dst, pad])
  src_t = src_p.reshape(NW, K_AGG, CH)
  dst_t = dst_p.reshape(NW, K_AGG, CH)
  deg_idx = jnp.stack([src_p, dst_p]).reshape(2, NS, K_DEG, CH)

  onesd = jnp.ones((CH, D), jnp.float32)
  zerosd = jnp.zeros((RPT, D), jnp.float32)
  x_pad = jnp.concatenate(
      [features, jnp.zeros((ROWS_SH - N, D), jnp.float32)])

  deg = _sc_degrees(deg_idx, onesd, zerosd)
  u1 = _tc_matmul(x_pad, W1)
  h1s, scales = _tc_scale_l1(u1, deg)
  p1 = _sc_aggregate(h1s, src_t, dst_t, zerosd)
  h2s = _tc_mid(p1, scales, b1.reshape(1, D), gamma1.reshape(1, D),
                beta1.reshape(1, D), W2)
  p2 = _sc_aggregate(h2s, src_t, dst_t, zerosd)
  return _tc_post(p2, scales, b2.reshape(1, D), gamma2.reshape(1, D),
                  beta2.reshape(1, D))


# speedup vs baseline: 2.6454x; 2.6454x over previous
import functools

import jax
import jax.numpy as jnp
from jax import lax
from jax.experimental import pallas as pl
from jax.experimental.pallas import tpu as pltpu
from jax.experimental.pallas import tpu_sc as plsc

N = 10000
D = 128
HD = D // 2
E = 320000
EPS = 1e-5

NC = 2
NS = 16
NW = NC * NS
CH = 128
K_AGG = 160
K_DEG = 160
E_PAD = NS * K_AGG * CH
ROWS_SH = 10112
RPT = ROWS_SH // NS
DEG_W = 16

_NO_TC_TILING = pltpu.CompilerParams(use_tc_tiling_on_sc=False)


def _sc_degrees(deg_idx, ones_hbm, zeros_hbm):
  mesh = plsc.VectorSubcoreMesh(core_axis_name="c", subcore_axis_name="s")

  @functools.partial(
      pl.kernel,
      out_type=jax.ShapeDtypeStruct((NC, ROWS_SH, DEG_W), jnp.float32),
      mesh=mesh,
      compiler_params=_NO_TC_TILING,
      scratch_types=[
          pltpu.VMEM((K_DEG, CH), jnp.int32),
          pltpu.VMEM((CH, DEG_W), jnp.float32),
          pltpu.VMEM_SHARED((ROWS_SH, DEG_W), jnp.float32),
      ],
  )
  def k(idx_hbm, ones_h, zeros_h, out_hbm, idx_v, ones_v, deg_sh):
    c = lax.axis_index("c")
    s = lax.axis_index("s")
    pltpu.sync_copy(idx_hbm.at[c, s], idx_v)
    pltpu.sync_copy(ones_h, ones_v)
    pltpu.sync_copy(zeros_h, deg_sh.at[pl.ds(s * RPT, RPT)])
    plsc.subcore_barrier()

    @pl.loop(0, K_DEG)
    def _(j):
      pltpu.sync_copy(ones_v, deg_sh.at[idx_v.at[j]], add=True)

    plsc.subcore_barrier()
    pltpu.sync_copy(deg_sh.at[pl.ds(s * RPT, RPT)],
                    out_hbm.at[c, pl.ds(s * RPT, RPT)])

  return k(deg_idx, ones_hbm, zeros_hbm)


def _sc_aggregate(h_half, src_t, dst_t, zeros_hbm):
  mesh = plsc.VectorSubcoreMesh(core_axis_name="c", subcore_axis_name="s")

  @functools.partial(
      pl.kernel,
      out_type=jax.ShapeDtypeStruct((NC, ROWS_SH, HD), jnp.float32),
      mesh=mesh,
      compiler_params=_NO_TC_TILING,
      scratch_types=[
          pltpu.VMEM((K_AGG // 2, CH), jnp.int32),
          pltpu.VMEM((K_AGG // 2, CH), jnp.int32),
          pltpu.VMEM((CH, HD), jnp.float32),
          pltpu.VMEM((CH, HD), jnp.float32),
          pltpu.VMEM_SHARED((ROWS_SH, HD), jnp.float32),
          pltpu.VMEM_SHARED((ROWS_SH, HD), jnp.float32),
          pltpu.SemaphoreType.DMA,
          pltpu.SemaphoreType.DMA,
      ],
  )
  def k(h_hbm, src_h, dst_h, z_h, out_hbm, sidx_v, didx_v, rows0, rows1,
        tab_sh, agg_sh, g0, g1):
    c = lax.axis_index("c")
    s = lax.axis_index("s")
    half = K_AGG // 2
    pltpu.sync_copy(h_hbm.at[c, pl.ds(s * RPT, RPT)],
                    tab_sh.at[pl.ds(s * RPT, RPT)])
    pltpu.sync_copy(z_h, agg_sh.at[pl.ds(s * RPT, RPT)])
    plsc.subcore_barrier()

    def wait_gather(buf, sem):
      pltpu.make_async_copy(tab_sh.at[sidx_v.at[0]], buf, sem).wait()

    for hf in range(2):
      pltpu.sync_copy(src_h.at[s, pl.ds(hf * half, half)], sidx_v)
      pltpu.sync_copy(dst_h.at[s, pl.ds(hf * half, half)], didx_v)
      pltpu.async_copy(tab_sh.at[sidx_v.at[0]], rows0, g0)

      @pl.loop(0, half - 2, step=2)
      def _(j):
        wait_gather(rows0, g0)
        pltpu.async_copy(tab_sh.at[sidx_v.at[j + 1]], rows1, g1)
        pltpu.sync_copy(rows0, agg_sh.at[didx_v.at[j]], add=True)
        wait_gather(rows1, g1)
        pltpu.async_copy(tab_sh.at[sidx_v.at[j + 2]], rows0, g0)
        pltpu.sync_copy(rows1, agg_sh.at[didx_v.at[j + 1]], add=True)

      wait_gather(rows0, g0)
      pltpu.async_copy(tab_sh.at[sidx_v.at[half - 1]], rows1, g1)
      pltpu.sync_copy(rows0, agg_sh.at[didx_v.at[half - 2]], add=True)
      wait_gather(rows1, g1)
      pltpu.sync_copy(rows1, agg_sh.at[didx_v.at[half - 1]], add=True)

    plsc.subcore_barrier()
    pltpu.sync_copy(agg_sh.at[pl.ds(s * RPT, RPT)],
                    out_hbm.at[c, pl.ds(s * RPT, RPT)])

  return k(h_half, src_t, dst_t, zeros_hbm)


def _tc_matmul(x_pad, w1):
  def body(x_ref, w_ref, o_ref):
    o_ref[...] = jnp.dot(x_ref[...], w_ref[...],
                         preferred_element_type=jnp.float32)

  return pl.pallas_call(
      body,
      out_shape=jax.ShapeDtypeStruct((ROWS_SH, D), jnp.float32),
  )(x_pad, w1)


def _tc_scale_l1(u, deg):
  def body(u_ref, d_ref, h_ref, s_ref):
    out_s = lax.rsqrt(jnp.maximum(d_ref[0, :, 0:1], 1.0))
    in_s = lax.rsqrt(jnp.maximum(d_ref[1, :, 0:1], 1.0))
    h = u_ref[...] * out_s
    h_ref[0, :, :] = h[:, 0:HD]
    h_ref[1, :, :] = h[:, HD:D]
    s_ref[...] = jnp.concatenate(
        [jnp.broadcast_to(out_s, (ROWS_SH, 8)),
         jnp.broadcast_to(in_s, (ROWS_SH, 8))], axis=1)

  return pl.pallas_call(
      body,
      out_shape=(jax.ShapeDtypeStruct((NC, ROWS_SH, HD), jnp.float32),
                 jax.ShapeDtypeStruct((ROWS_SH, 16), jnp.float32)),
  )(u, deg)


def _tc_mid(p, scales, b1, g1, be1, w2):
  def body(p_ref, s_ref, b_ref, g_ref, be_ref, w_ref, o_ref):
    agg = jnp.concatenate([p_ref[0, :N, :], p_ref[1, :N, :]], axis=1)
    agg = agg * s_ref[0:N, 8:9] + b_ref[...]
    a = jnp.where(agg > 0, agg, jnp.exp(agg) - 1.0)
    mean = jnp.mean(a, axis=0, keepdims=True)
    var = jnp.mean((a - mean) ** 2, axis=0, keepdims=True)
    h1 = (a - mean) * lax.rsqrt(var + EPS) * g_ref[...] + be_ref[...]
    h2 = jnp.dot(h1, w_ref[...], preferred_element_type=jnp.float32)
    h2 = h2 * s_ref[0:N, 0:1]
    pad = jnp.zeros((ROWS_SH - N, D), jnp.float32)
    h2p = jnp.concatenate([h2, pad], axis=0)
    o_ref[0, :, :] = h2p[:, 0:HD]
    o_ref[1, :, :] = h2p[:, HD:D]

  return pl.pallas_call(
      body,
      out_shape=jax.ShapeDtypeStruct((NC, ROWS_SH, HD), jnp.float32),
  )(p, scales, b1, g1, be1, w2)


def _tc_post(p, scales, b2, g2, be2):
  def body(p_ref, s_ref, b_ref, g_ref, be_ref, o_ref):
    agg = jnp.concatenate([p_ref[0, :N, :], p_ref[1, :N, :]], axis=1)
    agg = agg * s_ref[0:N, 8:9] + b_ref[...]
    a = jnp.where(agg > 0, agg, jnp.exp(agg) - 1.0)
    mean = jnp.mean(a, axis=0, keepdims=True)
    var = jnp.mean((a - mean) ** 2, axis=0, keepdims=True)
    o_ref[...] = (a - mean) * lax.rsqrt(var + EPS) * g_ref[...] + be_ref[...]

  return pl.pallas_call(
      body,
      out_shape=jax.ShapeDtypeStruct((N, D), jnp.float32),
  )(p, scales, b2, g2, be2)


@jax.jit
def kernel(features, edge_index, W1, b1, gamma1, beta1, W2, b2, gamma2,
           beta2):
  src = edge_index[0].astype(jnp.int32)
  dst = edge_index[1].astype(jnp.int32)
  pad = N + (jnp.arange(E_PAD - E, dtype=jnp.int32) % (ROWS_SH - N))
  src_p = jnp.concatenate([src, pad])
  dst_p = jnp.concatenate([dst, pad])
  src_t = src_p.reshape(NS, K_AGG, CH)
  dst_t = dst_p.reshape(NS, K_AGG, CH)
  deg_idx = jnp.stack([src_p, dst_p]).reshape(2, NS, K_DEG, CH)

  onesw = jnp.ones((CH, DEG_W), jnp.float32)
  zerosw = jnp.zeros((RPT, DEG_W), jnp.float32)
  zerosh = jnp.zeros((RPT, HD), jnp.float32)
  x_pad = jnp.concatenate(
      [features, jnp.zeros((ROWS_SH - N, D), jnp.float32)])

  deg = _sc_degrees(deg_idx, onesw, zerosw)
  u1 = _tc_matmul(x_pad, W1)
  h1s, scales = _tc_scale_l1(u1, deg)
  p1 = _sc_aggregate(h1s, src_t, dst_t, zerosh)
  h2s = _tc_mid(p1, scales, b1.reshape(1, D), gamma1.reshape(1, D),
                beta1.reshape(1, D), W2)
  p2 = _sc_aggregate(h2s, src_t, dst_t, zerosh)
  return _tc_post(p2, scales, b2.reshape(1, D), gamma2.reshape(1, D),
                  beta2.reshape(1, D))


# speedup vs baseline: 3.2344x; 1.2227x over previous
import functools

import jax
import jax.numpy as jnp
from jax import lax
from jax.experimental import pallas as pl
from jax.experimental.pallas import tpu as pltpu
from jax.experimental.pallas import tpu_sc as plsc

N = 10000
D = 128
E = 320000
EPS = 1e-5

NC = 2
NS = 16
NW = NC * NS
CH = 128
K_AGG = 80
K_DEG = 160
E_PAD = NW * K_AGG * CH
ROWS_SH = 10112
RPT = ROWS_SH // NS
DEG_W = 16


def _sc_degrees(deg_idx, ones_hbm, zeros_hbm):
  mesh = plsc.VectorSubcoreMesh(core_axis_name="c", subcore_axis_name="s")

  @functools.partial(
      pl.kernel,
      out_type=jax.ShapeDtypeStruct((NC, ROWS_SH, DEG_W), jnp.float32),
      mesh=mesh,
      compiler_params=pltpu.CompilerParams(use_tc_tiling_on_sc=False),
      scratch_types=[
          pltpu.VMEM((K_DEG, CH), jnp.int32),
          pltpu.VMEM((CH, DEG_W), jnp.float32),
          pltpu.VMEM_SHARED((ROWS_SH, DEG_W), jnp.float32),
          pltpu.SemaphoreType.DMA,
      ],
  )
  def k(idx_hbm, ones_h, zeros_h, out_hbm, idx_v, ones_v, deg_sh, sem):
    c = lax.axis_index("c")
    s = lax.axis_index("s")
    pltpu.sync_copy(idx_hbm.at[c, s], idx_v)
    pltpu.sync_copy(ones_h, ones_v)
    pltpu.sync_copy(zeros_h, deg_sh.at[pl.ds(s * RPT, RPT)])
    plsc.subcore_barrier()

    WIN = 4
    for w in range(WIN):
      pltpu.async_copy(ones_v, deg_sh.at[idx_v.at[w]], sem, add=True)

    @pl.loop(WIN, K_DEG)
    def _(j):
      pltpu.make_async_copy(ones_v, deg_sh.at[idx_v.at[0]], sem).wait()
      pltpu.async_copy(ones_v, deg_sh.at[idx_v.at[j]], sem, add=True)

    for w in range(WIN):
      pltpu.make_async_copy(ones_v, deg_sh.at[idx_v.at[0]], sem).wait()

    plsc.subcore_barrier()
    pltpu.sync_copy(deg_sh.at[pl.ds(s * RPT, RPT)],
                    out_hbm.at[c, pl.ds(s * RPT, RPT)])

  return k(deg_idx, ones_hbm, zeros_hbm)


def _sc_aggregate(h_pad, src_t, dst_t, zeros_hbm):
  mesh = plsc.VectorSubcoreMesh(core_axis_name="c", subcore_axis_name="s")

  @functools.partial(
      pl.kernel,
      out_type=jax.ShapeDtypeStruct((NC, ROWS_SH, D), jnp.float32),
      mesh=mesh,
      scratch_types=[
          pltpu.VMEM((K_AGG // 2, CH), jnp.int32),
          pltpu.VMEM((K_AGG // 2, CH), jnp.int32),
          pltpu.VMEM((CH, D), jnp.float32),
          pltpu.VMEM((CH, D), jnp.float32),
          pltpu.VMEM_SHARED((ROWS_SH, D), jnp.float32),
          pltpu.SemaphoreType.DMA,
          pltpu.SemaphoreType.DMA,
      ],
  )
  def k(h_hbm, src_h, dst_h, z_h, out_hbm, sidx_v, didx_v, rows0, rows1,
        agg_sh, g0, g1):
    c = lax.axis_index("c")
    s = lax.axis_index("s")
    wid = c * NS + s
    half = K_AGG // 2
    pltpu.sync_copy(z_h, agg_sh.at[pl.ds(s * RPT, RPT)])
    plsc.subcore_barrier()

    def wait_gather(buf, sem):
      pltpu.make_async_copy(h_hbm.at[sidx_v.at[0]], buf, sem).wait()

    for hf in range(2):
      pltpu.sync_copy(src_h.at[wid, pl.ds(hf * half, half)], sidx_v)
      pltpu.sync_copy(dst_h.at[wid, pl.ds(hf * half, half)], didx_v)
      pltpu.async_copy(h_hbm.at[sidx_v.at[0]], rows0, g0)

      @pl.loop(0, half - 2, step=2)
      def _(j):
        wait_gather(rows0, g0)
        pltpu.async_copy(h_hbm.at[sidx_v.at[j + 1]], rows1, g1)
        pltpu.sync_copy(rows0, agg_sh.at[didx_v.at[j]], add=True)
        wait_gather(rows1, g1)
        pltpu.async_copy(h_hbm.at[sidx_v.at[j + 2]], rows0, g0)
        pltpu.sync_copy(rows1, agg_sh.at[didx_v.at[j + 1]], add=True)

      wait_gather(rows0, g0)
      pltpu.async_copy(h_hbm.at[sidx_v.at[half - 1]], rows1, g1)
      pltpu.sync_copy(rows0, agg_sh.at[didx_v.at[half - 2]], add=True)
      wait_gather(rows1, g1)
      pltpu.sync_copy(rows1, agg_sh.at[didx_v.at[half - 1]], add=True)

    plsc.subcore_barrier()
    pltpu.sync_copy(agg_sh.at[pl.ds(s * RPT, RPT)],
                    out_hbm.at[c, pl.ds(s * RPT, RPT)])

  return k(h_pad, src_t, dst_t, zeros_hbm)


def _tc_matmul(x_pad, w1):
  def body(x_ref, w_ref, o_ref):
    o_ref[...] = jnp.dot(x_ref[...], w_ref[...],
                         preferred_element_type=jnp.float32)

  return pl.pallas_call(
      body,
      out_shape=jax.ShapeDtypeStruct((ROWS_SH, D), jnp.float32),
  )(x_pad, w1)


def _tc_scale_l1(u, deg):
  def body(u_ref, d_ref, h_ref, s_ref):
    out_s = lax.rsqrt(jnp.maximum(d_ref[0, :, 0:1], 1.0))
    in_s = lax.rsqrt(jnp.maximum(d_ref[1, :, 0:1], 1.0))
    h_ref[...] = u_ref[...] * out_s
    s_ref[...] = jnp.concatenate(
        [jnp.broadcast_to(out_s, (ROWS_SH, 8)),
         jnp.broadcast_to(in_s, (ROWS_SH, 8))], axis=1)

  return pl.pallas_call(
      body,
      out_shape=(jax.ShapeDtypeStruct((ROWS_SH, D), jnp.float32),
                 jax.ShapeDtypeStruct((ROWS_SH, 16), jnp.float32)),
  )(u, deg)


def _tc_mid(p, scales, b1, g1, be1, w2):
  def body(p_ref, s_ref, b_ref, g_ref, be_ref, w_ref, o_ref):
    agg = p_ref[0, :N, :] + p_ref[1, :N, :]
    agg = agg * s_ref[0:N, 8:9] + b_ref[...]
    a = jnp.where(agg > 0, agg, jnp.exp(agg) - 1.0)
    mean = jnp.mean(a, axis=0, keepdims=True)
    var = jnp.mean((a - mean) ** 2, axis=0, keepdims=True)
    h1 = (a - mean) * lax.rsqrt(var + EPS) * g_ref[...] + be_ref[...]
    h2 = jnp.dot(h1, w_ref[...], preferred_element_type=jnp.float32)
    o_ref[0:N, :] = h2 * s_ref[0:N, 0:1]
    o_ref[N:ROWS_SH, :] = jnp.zeros((ROWS_SH - N, D), jnp.float32)

  return pl.pallas_call(
      body,
      out_shape=jax.ShapeDtypeStruct((ROWS_SH, D), jnp.float32),
  )(p, scales, b1, g1, be1, w2)


def _tc_post(p, scales, b2, g2, be2):
  def body(p_ref, s_ref, b_ref, g_ref, be_ref, o_ref):
    agg = p_ref[0, :N, :] + p_ref[1, :N, :]
    agg = agg * s_ref[0:N, 8:9] + b_ref[...]
    a = jnp.where(agg > 0, agg, jnp.exp(agg) - 1.0)
    mean = jnp.mean(a, axis=0, keepdims=True)
    var = jnp.mean((a - mean) ** 2, axis=0, keepdims=True)
    o_ref[...] = (a - mean) * lax.rsqrt(var + EPS) * g_ref[...] + be_ref[...]

  return pl.pallas_call(
      body,
      out_shape=jax.ShapeDtypeStruct((N, D), jnp.float32),
  )(p, scales, b2, g2, be2)


@jax.jit
def kernel(features, edge_index, W1, b1, gamma1, beta1, W2, b2, gamma2,
           beta2):
  src = edge_index[0].astype(jnp.int32)
  dst = edge_index[1].astype(jnp.int32)
  pad = N + (jnp.arange(E_PAD - E, dtype=jnp.int32) % (ROWS_SH - N))
  src_p = jnp.concatenate([src, pad])
  dst_p = jnp.concatenate([dst, pad])
  src_t = src_p.reshape(NW, K_AGG, CH)
  dst_t = dst_p.reshape(NW, K_AGG, CH)
  deg_idx = jnp.stack([src_p, dst_p]).reshape(2, NS, K_DEG, CH)

  onesw = jnp.ones((CH, DEG_W), jnp.float32)
  zerosw = jnp.zeros((RPT, DEG_W), jnp.float32)
  zerosd = jnp.zeros((RPT, D), jnp.float32)
  x_pad = jnp.concatenate(
      [features, jnp.zeros((ROWS_SH - N, D), jnp.float32)])

  deg = _sc_degrees(deg_idx, onesw, zerosw)
  u1 = _tc_matmul(x_pad, W1)
  h1s, scales = _tc_scale_l1(u1, deg)
  p1 = _sc_aggregate(h1s, src_t, dst_t, zerosd)
  h2s = _tc_mid(p1, scales, b1.reshape(1, D), gamma1.reshape(1, D),
                beta1.reshape(1, D), W2)
  p2 = _sc_aggregate(h2s, src_t, dst_t, zerosd)
  return _tc_post(p2, scales, b2.reshape(1, D), gamma2.reshape(1, D),
                  beta2.reshape(1, D))


# speedup vs baseline: 3.2482x; 1.0043x over previous
import functools

import jax
import jax.numpy as jnp
from jax import lax
from jax.experimental import pallas as pl
from jax.experimental.pallas import tpu as pltpu
from jax.experimental.pallas import tpu_sc as plsc

N = 10000
D = 128
E = 320000
EPS = 1e-5

NC = 2
NS = 16
NW = NC * NS
CH = 128
K_AGG = 80
K_DEG = 160
E_PAD = NW * K_AGG * CH
ROWS_SH = 10112
RPT = ROWS_SH // NS
DEG_W = 16


def _sc_degrees(deg_idx, ones_hbm, zeros_hbm):
  mesh = plsc.VectorSubcoreMesh(core_axis_name="c", subcore_axis_name="s")

  @functools.partial(
      pl.kernel,
      out_type=jax.ShapeDtypeStruct((NC, ROWS_SH, DEG_W), jnp.float32),
      mesh=mesh,
      compiler_params=pltpu.CompilerParams(use_tc_tiling_on_sc=False),
      scratch_types=[
          pltpu.VMEM((K_DEG, CH), jnp.int32),
          pltpu.VMEM((CH, DEG_W), jnp.float32),
          pltpu.VMEM_SHARED((ROWS_SH, DEG_W), jnp.float32),
          pltpu.SemaphoreType.DMA,
      ],
  )
  def k(idx_hbm, ones_h, zeros_h, out_hbm, idx_v, ones_v, deg_sh, sem):
    c = lax.axis_index("c")
    s = lax.axis_index("s")
    pltpu.sync_copy(idx_hbm.at[c, s], idx_v)
    pltpu.sync_copy(ones_h, ones_v)
    pltpu.sync_copy(zeros_h, deg_sh.at[pl.ds(s * RPT, RPT)])
    plsc.subcore_barrier()

    WIN = 4
    for w in range(WIN):
      pltpu.async_copy(ones_v, deg_sh.at[idx_v.at[w]], sem, add=True)

    @pl.loop(WIN, K_DEG)
    def _(j):
      pltpu.make_async_copy(ones_v, deg_sh.at[idx_v.at[0]], sem).wait()
      pltpu.async_copy(ones_v, deg_sh.at[idx_v.at[j]], sem, add=True)

    for w in range(WIN):
      pltpu.make_async_copy(ones_v, deg_sh.at[idx_v.at[0]], sem).wait()

    plsc.subcore_barrier()
    pltpu.sync_copy(deg_sh.at[pl.ds(s * RPT, RPT)],
                    out_hbm.at[c, pl.ds(s * RPT, RPT)])

  return k(deg_idx, ones_hbm, zeros_hbm)


def _sc_aggregate(h_pad, src_t, dst_t, zeros_hbm):
  mesh = plsc.VectorSubcoreMesh(core_axis_name="c", subcore_axis_name="s")

  @functools.partial(
      pl.kernel,
      out_type=jax.ShapeDtypeStruct((NC, ROWS_SH, D), jnp.float32),
      mesh=mesh,
      scratch_types=[
          pltpu.VMEM((K_AGG // 2, CH), jnp.int32),
          pltpu.VMEM((K_AGG // 2, CH), jnp.int32),
          pltpu.VMEM((CH, D), jnp.float32),
          pltpu.VMEM((CH, D), jnp.float32),
          pltpu.VMEM_SHARED((ROWS_SH, D), jnp.float32),
          pltpu.SemaphoreType.DMA,
          pltpu.SemaphoreType.DMA,
      ],
  )
  def k(h_hbm, src_h, dst_h, z_h, out_hbm, sidx_v, didx_v, rows0, rows1,
        agg_sh, g0, g1):
    c = lax.axis_index("c")
    s = lax.axis_index("s")
    wid = c * NS + s
    half = K_AGG // 2
    pltpu.sync_copy(z_h, agg_sh.at[pl.ds(s * RPT, RPT)])
    plsc.subcore_barrier()

    def wait_gather(buf, sem):
      pltpu.make_async_copy(h_hbm.at[sidx_v.at[0]], buf, sem).wait()

    for hf in range(2):
      pltpu.sync_copy(src_h.at[wid, pl.ds(hf * half, half)], sidx_v)
      pltpu.sync_copy(dst_h.at[wid, pl.ds(hf * half, half)], didx_v)
      pltpu.async_copy(h_hbm.at[sidx_v.at[0]], rows0, g0)

      @pl.loop(0, half - 2, step=2)
      def _(j):
        wait_gather(rows0, g0)
        pltpu.async_copy(h_hbm.at[sidx_v.at[j + 1]], rows1, g1)
        pltpu.sync_copy(rows0, agg_sh.at[didx_v.at[j]], add=True)
        wait_gather(rows1, g1)
        pltpu.async_copy(h_hbm.at[sidx_v.at[j + 2]], rows0, g0)
        pltpu.sync_copy(rows1, agg_sh.at[didx_v.at[j + 1]], add=True)

      wait_gather(rows0, g0)
      pltpu.async_copy(h_hbm.at[sidx_v.at[half - 1]], rows1, g1)
      pltpu.sync_copy(rows0, agg_sh.at[didx_v.at[half - 2]], add=True)
      wait_gather(rows1, g1)
      pltpu.sync_copy(rows1, agg_sh.at[didx_v.at[half - 1]], add=True)

    plsc.subcore_barrier()
    pltpu.sync_copy(agg_sh.at[pl.ds(s * RPT, RPT)],
                    out_hbm.at[c, pl.ds(s * RPT, RPT)])

  return k(h_pad, src_t, dst_t, zeros_hbm)


def _tc_pre(x_pad, w1, deg):
  def body(x_ref, w_ref, d_ref, h_ref, s_ref):
    u = jnp.dot(x_ref[...], w_ref[...], preferred_element_type=jnp.float32)
    out_s = lax.rsqrt(jnp.maximum(d_ref[0, :, 0:1], 1.0))
    in_s = lax.rsqrt(jnp.maximum(d_ref[1, :, 0:1], 1.0))
    h_ref[...] = u * out_s
    s_ref[...] = jnp.concatenate(
        [jnp.broadcast_to(out_s, (ROWS_SH, 8)),
         jnp.broadcast_to(in_s, (ROWS_SH, 8))], axis=1)

  return pl.pallas_call(
      body,
      out_shape=(jax.ShapeDtypeStruct((ROWS_SH, D), jnp.float32),
                 jax.ShapeDtypeStruct((ROWS_SH, 16), jnp.float32)),
  )(x_pad, w1, deg)


def _tc_mid(p, scales, b1, g1, be1, w2):
  def body(p_ref, s_ref, b_ref, g_ref, be_ref, w_ref, o_ref):
    agg = p_ref[0, :N, :] + p_ref[1, :N, :]
    agg = agg * s_ref[0:N, 8:9] + b_ref[...]
    a = jnp.where(agg > 0, agg, jnp.exp(agg) - 1.0)
    mean = jnp.mean(a, axis=0, keepdims=True)
    var = jnp.mean((a - mean) ** 2, axis=0, keepdims=True)
    h1 = (a - mean) * lax.rsqrt(var + EPS) * g_ref[...] + be_ref[...]
    h2 = jnp.dot(h1, w_ref[...], preferred_element_type=jnp.float32)
    o_ref[0:N, :] = h2 * s_ref[0:N, 0:1]
    o_ref[N:ROWS_SH, :] = jnp.zeros((ROWS_SH - N, D), jnp.float32)

  return pl.pallas_call(
      body,
      out_shape=jax.ShapeDtypeStruct((ROWS_SH, D), jnp.float32),
  )(p, scales, b1, g1, be1, w2)


def _tc_post(p, scales, b2, g2, be2):
  def body(p_ref, s_ref, b_ref, g_ref, be_ref, o_ref):
    agg = p_ref[0, :N, :] + p_ref[1, :N, :]
    agg = agg * s_ref[0:N, 8:9] + b_ref[...]
    a = jnp.where(agg > 0, agg, jnp.exp(agg) - 1.0)
    mean = jnp.mean(a, axis=0, keepdims=True)
    var = jnp.mean((a - mean) ** 2, axis=0, keepdims=True)
    o_ref[...] = (a - mean) * lax.rsqrt(var + EPS) * g_ref[...] + be_ref[...]

  return pl.pallas_call(
      body,
      out_shape=jax.ShapeDtypeStruct((N, D), jnp.float32),
  )(p, scales, b2, g2, be2)


@jax.jit
def kernel(features, edge_index, W1, b1, gamma1, beta1, W2, b2, gamma2,
           beta2):
  src = edge_index[0].astype(jnp.int32)
  dst = edge_index[1].astype(jnp.int32)
  pad = N + (jnp.arange(E_PAD - E, dtype=jnp.int32) % (ROWS_SH - N))
  src_p = jnp.concatenate([src, pad])
  dst_p = jnp.concatenate([dst, pad])
  src_t = src_p.reshape(NW, K_AGG, CH)
  dst_t = dst_p.reshape(NW, K_AGG, CH)
  deg_idx = jnp.stack([src_p, dst_p]).reshape(2, NS, K_DEG, CH)

  onesw = jnp.ones((CH, DEG_W), jnp.float32)
  zerosw = jnp.zeros((RPT, DEG_W), jnp.float32)
  zerosd = jnp.zeros((RPT, D), jnp.float32)
  x_pad = jnp.concatenate(
      [features, jnp.zeros((ROWS_SH - N, D), jnp.float32)])

  deg = _sc_degrees(deg_idx, onesw, zerosw)
  h1s, scales = _tc_pre(x_pad, W1, deg)
  p1 = _sc_aggregate(h1s, src_t, dst_t, zerosd)
  h2s = _tc_mid(p1, scales, b1.reshape(1, D), gamma1.reshape(1, D),
                beta1.reshape(1, D), W2)
  p2 = _sc_aggregate(h2s, src_t, dst_t, zerosd)
  return _tc_post(p2, scales, b2.reshape(1, D), gamma2.reshape(1, D),
                  beta2.reshape(1, D))
